# baseline (speedup 1.0000x reference)
import jax
import jax.numpy as jnp
from jax import lax
from jax.experimental import pallas as pl
from jax.experimental.pallas import tpu as pltpu
from jax.experimental.pallas import tpu_sc as plsc

MAX_F = 64
_NC, _NS, _L = 2, 16, 16
_NW = _NC * _NS
_TR = 512 // _NW
_NB = 4
_SLABS = 256


def _sc_body(x_hbm, f_hbm, t_hbm, o_hbm, freq_v, time_v, in_b, out_b, in_s, out_s):
    c = lax.axis_index("c")
    sid = lax.axis_index("s")
    wid = sid * _NC + c
    t_lo = wid * _TR
    pltpu.sync_copy(f_hbm, freq_v)
    pltpu.sync_copy(t_hbm.at[pl.ds(t_lo, _TR)], time_v)

    def in_copy(slab, b):
        return pltpu.make_async_copy(
            x_hbm.at[slab, pl.ds(t_lo, _TR), :], in_b.at[b], in_s.at[b]
        )

    def out_copy(slab, b):
        return pltpu.make_async_copy(
            out_b.at[b], o_hbm.at[slab, pl.ds(t_lo, _TR), :], out_s.at[b]
        )

    for b in range(_NB):
        in_copy(b, b).start()

    n_groups = _SLABS // _NB

    def group(g, _):
        for b in range(_NB):
            slab = g * _NB + b
            in_copy(slab, b).wait()

            @pl.when(g > 0)
            def _():
                out_copy(slab - _NB, b).wait()

            f0 = lax.rem(slab, MAX_F)

            def row(r, _2):
                for j in range(16):
                    sl = pl.ds(j * _L, _L)
                    out_b[b, r, sl] = in_b[b, r, sl] + freq_v[f0, sl]
                for j in range(16):
                    slx = pl.ds(256 + j * _L, _L)
                    slt = pl.ds(j * _L, _L)
                    out_b[b, r, slx] = in_b[b, r, slx] + time_v[r, slt]
                return 0

            lax.fori_loop(0, _TR, row, 0)
            out_copy(slab, b).start()

            @pl.when(g + 1 < n_groups)
            def _():
                in_copy(slab + _NB, b).start()

        return 0

    lax.fori_loop(0, n_groups, group, 0)
    for b in range(_NB):
        out_copy(_SLABS - _NB + b, b).wait()


def kernel(x, freq_embed, time_embed):
    B, F, T, D = x.shape
    xf = x.reshape(B * F, T, D)
    out = pl.kernel(
        _sc_body,
        out_type=jax.ShapeDtypeStruct(xf.shape, x.dtype),
        mesh=plsc.VectorSubcoreMesh(core_axis_name="c", subcore_axis_name="s"),
        scratch_types=[
            pltpu.VMEM((MAX_F, 256), jnp.float32),
            pltpu.VMEM((_TR, 256), jnp.float32),
            pltpu.VMEM((_NB, _TR, 512), jnp.float32),
            pltpu.VMEM((_NB, _TR, 512), jnp.float32),
            pltpu.SemaphoreType.DMA((_NB,)),
            pltpu.SemaphoreType.DMA((_NB,)),
        ],
    )(xf, freq_embed, time_embed)
    return out.reshape(B, F, T, D)


# speedup vs baseline: 2.6056x; 2.6056x over previous
import jax
import jax.numpy as jnp
from jax import lax
from jax.experimental import pallas as pl
from jax.experimental.pallas import tpu as pltpu
from jax.experimental.pallas import tpu_sc as plsc

MAX_F = 64
_NC, _NS, _L = 2, 16, 16
_NW = _NC * _NS
_TR = 512 // _NW
_NB = 2
_SLABS = 256


def _sc_body(x_hbm, f_hbm, t_hbm, o_hbm, freq_v, time_v, in_b, out_b, in_s, out_s):
    c = lax.axis_index("c")
    sid = lax.axis_index("s")
    wid = sid * _NC + c
    t_lo = wid * _TR
    pltpu.sync_copy(f_hbm, freq_v)
    pltpu.sync_copy(t_hbm.at[pl.ds(t_lo, _TR)], time_v)

    def in_copy(slab, b):
        return pltpu.make_async_copy(
            x_hbm.at[slab, pl.ds(t_lo, _TR), :], in_b.at[b], in_s.at[b]
        )

    def out_copy(slab, b):
        return pltpu.make_async_copy(
            out_b.at[b], o_hbm.at[slab, pl.ds(t_lo, _TR), :], out_s.at[b]
        )

    for b in range(_NB):
        in_copy(b, b).start()

    def step(slab, _):
        b = lax.rem(slab, _NB)
        in_copy(slab, b).wait()

        @pl.when(slab >= _NB)
        def _():
            out_copy(slab - _NB, b).wait()

        f0 = lax.rem(slab, MAX_F)
        fvec = [freq_v[f0, pl.ds(j * _L, _L)] for j in range(16)]
        for r in range(_TR):
            xa = [in_b[b, r, pl.ds(j * _L, _L)] for j in range(16)]
            for j in range(16):
                out_b[b, r, pl.ds(j * _L, _L)] = xa[j] + fvec[j]
            xb = [in_b[b, r, pl.ds(256 + j * _L, _L)] for j in range(16)]
            tv = [time_v[r, pl.ds(j * _L, _L)] for j in range(16)]
            for j in range(16):
                out_b[b, r, pl.ds(256 + j * _L, _L)] = xb[j] + tv[j]

        out_copy(slab, b).start()

        @pl.when(slab + _NB < _SLABS)
        def _():
            in_copy(slab + _NB, b).start()

        return 0

    lax.fori_loop(0, _SLABS, step, 0)
    for b in range(_NB):
        out_copy(_SLABS - _NB + b, b).wait()


def kernel(x, freq_embed, time_embed):
    B, F, T, D = x.shape
    xf = x.reshape(B * F, T, D)
    out = pl.kernel(
        _sc_body,
        out_type=jax.ShapeDtypeStruct(xf.shape, x.dtype),
        mesh=plsc.VectorSubcoreMesh(core_axis_name="c", subcore_axis_name="s"),
        scratch_types=[
            pltpu.VMEM((MAX_F, 256), jnp.float32),
            pltpu.VMEM((_TR, 256), jnp.float32),
            pltpu.VMEM((_NB, _TR, 512), jnp.float32),
            pltpu.VMEM((_NB, _TR, 512), jnp.float32),
            pltpu.SemaphoreType.DMA((_NB,)),
            pltpu.SemaphoreType.DMA((_NB,)),
        ],
    )(xf, freq_embed, time_embed)
    return out.reshape(B, F, T, D)


# speedup vs baseline: 3.1619x; 1.2135x over previous
import jax
import jax.numpy as jnp
from jax import lax
from jax.experimental import pallas as pl
from jax.experimental.pallas import tpu as pltpu
from jax.experimental.pallas import tpu_sc as plsc

MAX_F = 64
_NC, _NS, _L = 2, 16, 16
_NW = _NC * _NS
_TR = 512 // _NW
_NB = 4
_SLABS = 256


def _sc_body(x_hbm, f_hbm, t_hbm, o_hbm, freq_v, time_v, in_b, out_b, in_s, out_s):
    c = lax.axis_index("c")
    sid = lax.axis_index("s")
    wid = sid * _NC + c
    t_lo = wid * _TR
    pltpu.sync_copy(f_hbm, freq_v)
    pltpu.sync_copy(t_hbm.at[pl.ds(t_lo, _TR)], time_v)

    def in_copy(slab, b):
        return pltpu.make_async_copy(
            x_hbm.at[slab, pl.ds(t_lo, _TR), :], in_b.at[b], in_s.at[b]
        )

    def out_copy(slab, b):
        return pltpu.make_async_copy(
            out_b.at[b], o_hbm.at[slab, pl.ds(t_lo, _TR), :], out_s.at[b]
        )

    for b in range(_NB):
        in_copy(b, b).start()

    def step(slab, _):
        b = lax.rem(slab, _NB)
        in_copy(slab, b).wait()

        @pl.when(slab >= _NB)
        def _():
            out_copy(slab - _NB, b).wait()

        f0 = lax.rem(slab, MAX_F)
        fvec = [freq_v[f0, pl.ds(j * _L, _L)] for j in range(16)]
        for r in range(_TR):
            xa = [in_b[b, r, pl.ds(j * _L, _L)] for j in range(16)]
            for j in range(16):
                out_b[b, r, pl.ds(j * _L, _L)] = xa[j] + fvec[j]
            xb = [in_b[b, r, pl.ds(256 + j * _L, _L)] for j in range(16)]
            tv = [time_v[r, pl.ds(j * _L, _L)] for j in range(16)]
            for j in range(16):
                out_b[b, r, pl.ds(256 + j * _L, _L)] = xb[j] + tv[j]

        out_copy(slab, b).start()

        @pl.when(slab + _NB < _SLABS)
        def _():
            in_copy(slab + _NB, b).start()

        return 0

    lax.fori_loop(0, _SLABS, step, 0)
    for b in range(_NB):
        out_copy(_SLABS - _NB + b, b).wait()


def kernel(x, freq_embed, time_embed):
    B, F, T, D = x.shape
    xf = x.reshape(B * F, T, D)
    out = pl.kernel(
        _sc_body,
        out_type=jax.ShapeDtypeStruct(xf.shape, x.dtype),
        mesh=plsc.VectorSubcoreMesh(core_axis_name="c", subcore_axis_name="s"),
        scratch_types=[
            pltpu.VMEM((MAX_F, 256), jnp.float32),
            pltpu.VMEM((_TR, 256), jnp.float32),
            pltpu.VMEM((_NB, _TR, 512), jnp.float32),
            pltpu.VMEM((_NB, _TR, 512), jnp.float32),
            pltpu.SemaphoreType.DMA((_NB,)),
            pltpu.SemaphoreType.DMA((_NB,)),
        ],
    )(xf, freq_embed, time_embed)
    return out.reshape(B, F, T, D)


# speedup vs baseline: 3.1735x; 1.0037x over previous
import jax
import jax.numpy as jnp
from jax import lax
from jax.experimental import pallas as pl
from jax.experimental.pallas import tpu as pltpu
from jax.experimental.pallas import tpu_sc as plsc

MAX_F = 64
_NC, _NS, _L = 2, 16, 16
_NW = _NC * _NS
_TR = 512 // _NW
_NB = 6
_SLABS = 256


def _sc_body(x_hbm, f_hbm, t_hbm, o_hbm, freq_v, time_v, in_b, out_b, in_s, out_s):
    c = lax.axis_index("c")
    sid = lax.axis_index("s")
    wid = sid * _NC + c
    t_lo = wid * _TR
    pltpu.sync_copy(f_hbm, freq_v)
    pltpu.sync_copy(t_hbm.at[pl.ds(t_lo, _TR)], time_v)

    def in_copy(slab, b):
        return pltpu.make_async_copy(
            x_hbm.at[slab, pl.ds(t_lo, _TR), :], in_b.at[b], in_s.at[b]
        )

    def out_copy(slab, b):
        return pltpu.make_async_copy(
            out_b.at[b], o_hbm.at[slab, pl.ds(t_lo, _TR), :], out_s.at[b]
        )

    for b in range(_NB):
        in_copy(b, b).start()

    def step(slab, _):
        b = lax.rem(slab, _NB)
        in_copy(slab, b).wait()

        @pl.when(slab >= _NB)
        def _():
            out_copy(slab - _NB, b).wait()

        f0 = lax.rem(slab, MAX_F)
        fvec = [freq_v[f0, pl.ds(j * _L, _L)] for j in range(16)]
        for r in range(_TR):
            xa = [in_b[b, r, pl.ds(j * _L, _L)] for j in range(16)]
            for j in range(16):
                out_b[b, r, pl.ds(j * _L, _L)] = xa[j] + fvec[j]
            xb = [in_b[b, r, pl.ds(256 + j * _L, _L)] for j in range(16)]
            tv = [time_v[r, pl.ds(j * _L, _L)] for j in range(16)]
            for j in range(16):
                out_b[b, r, pl.ds(256 + j * _L, _L)] = xb[j] + tv[j]

        out_copy(slab, b).start()

        @pl.when(slab + _NB < _SLABS)
        def _():
            in_copy(slab + _NB, b).start()

        return 0

    lax.fori_loop(0, _SLABS, step, 0)
    for b in range(_NB):
        out_copy(_SLABS - _NB + b, b).wait()


def kernel(x, freq_embed, time_embed):
    B, F, T, D = x.shape
    xf = x.reshape(B * F, T, D)
    out = pl.kernel(
        _sc_body,
        out_type=jax.ShapeDtypeStruct(xf.shape, x.dtype),
        mesh=plsc.VectorSubcoreMesh(core_axis_name="c", subcore_axis_name="s"),
        scratch_types=[
            pltpu.VMEM((MAX_F, 256), jnp.float32),
            pltpu.VMEM((_TR, 256), jnp.float32),
            pltpu.VMEM((_NB, _TR, 512), jnp.float32),
            pltpu.VMEM((_NB, _TR, 512), jnp.float32),
            pltpu.SemaphoreType.DMA((_NB,)),
            pltpu.SemaphoreType.DMA((_NB,)),
        ],
    )(xf, freq_embed, time_embed)
    return out.reshape(B, F, T, D)
